# baseline (speedup 1.0000x reference)
import functools

import jax
import jax.numpy as jnp
from jax import lax
from jax.experimental import pallas as pl
from jax.experimental.pallas import tpu as pltpu
from jax.experimental.pallas import tpu_sc as plsc

NUM_SC_CORES = 2
NUM_SUBCORES = 16
NUM_WORKERS = NUM_SC_CORES * NUM_SUBCORES

BATCH = 16384
EMBED = 32
ROWS_PER_WORKER = BATCH // NUM_WORKERS
IDX_CHUNK = 128
NUM_CHUNKS = ROWS_PER_WORKER // IDX_CHUNK


def _sc_gather_one(table_pad, ids):
    mesh = plsc.VectorSubcoreMesh(core_axis_name="c", subcore_axis_name="s")

    @functools.partial(
        pl.kernel,
        mesh=mesh,
        compiler_params=pltpu.CompilerParams(use_tc_tiling_on_sc=False),
        out_type=jax.ShapeDtypeStruct((BATCH, 128), jnp.bfloat16),
        scratch_types=[
            pltpu.VMEM((ROWS_PER_WORKER,), jnp.int32),
            pltpu.VMEM((ROWS_PER_WORKER, 128), jnp.bfloat16),
            pltpu.SemaphoreType.DMA,
        ],
    )
    def k(t_hbm, id_hbm, out, idx_v, rows_v, sem):
        wid = lax.axis_index("s") * NUM_SC_CORES + lax.axis_index("c")
        base = wid * ROWS_PER_WORKER
        pltpu.sync_copy(id_hbm.at[pl.ds(base, ROWS_PER_WORKER)], idx_v)
        copies = []
        for j in range(NUM_CHUNKS):
            idx = pl.ds(j * IDX_CHUNK, IDX_CHUNK)
            copies.append(pltpu.async_copy(
                t_hbm.at[idx_v.at[idx]], rows_v.at[idx], sem))
        for c in copies:
            c.wait()
        pltpu.sync_copy(rows_v, out.at[pl.ds(base, ROWS_PER_WORKER)])

    return k(table_pad, ids)


def _tc_body(u_ref, i_ref, f_ref, wf_ref, bf_ref, w_ref, b_ref, out_ref):
    g = lax.dot_general(f_ref[...], wf_ref[...], (((1,), (1,)), ((), ())),
                        preferred_element_type=jnp.float32)
    u = u_ref[...][:, :EMBED].astype(jnp.float32)
    i = i_ref[...][:, :EMBED].astype(jnp.float32)
    acc = u * (i + g + bf_ref[...])
    out = jnp.sum(acc * w_ref[...], axis=1) + b_ref[0, 0]
    out_ref[...] = out[None, None, :]


def _tc_combine(u4, i4, f, Wf, bf2, W, b2):
    nblk = 8
    blk = BATCH // nblk
    return pl.pallas_call(
        _tc_body,
        grid=(nblk,),
        in_specs=[
            pl.BlockSpec((blk, 128), lambda n: (n, 0)),
            pl.BlockSpec((blk, 128), lambda n: (n, 0)),
            pl.BlockSpec((blk, 100), lambda n: (n, 0)),
            pl.BlockSpec((EMBED, 100), lambda n: (0, 0)),
            pl.BlockSpec((1, EMBED), lambda n: (0, 0)),
            pl.BlockSpec((1, EMBED), lambda n: (0, 0)),
            pl.BlockSpec((1, 1), lambda n: (0, 0)),
        ],
        out_specs=pl.BlockSpec((1, 1, blk), lambda n: (n, 0, 0)),
        out_shape=jax.ShapeDtypeStruct((nblk, 1, blk), jnp.float32),
    )(u4, i4, f, Wf, bf2, W, b2)


def kernel(user_ids, item_ids, features, user_table, item_table, Wf, bf, W, b):
    ut_pad = jnp.pad(user_table.astype(jnp.bfloat16),
                     ((0, 0), (0, 128 - EMBED)))
    it_pad = jnp.pad(item_table.astype(jnp.bfloat16),
                     ((0, 0), (0, 128 - EMBED)))
    u = _sc_gather_one(ut_pad, user_ids.astype(jnp.int32))
    i = _sc_gather_one(it_pad, item_ids.astype(jnp.int32))
    out = _tc_combine(u, i, features, Wf, bf.reshape(1, EMBED), W,
                      b.reshape(1, 1))
    return out.reshape(BATCH)


# speedup vs baseline: 2.3546x; 2.3546x over previous
import functools

import jax
import jax.numpy as jnp
from jax import lax
from jax.experimental import pallas as pl
from jax.experimental.pallas import tpu as pltpu
from jax.experimental.pallas import tpu_sc as plsc

NUM_SC_CORES = 2
NUM_SUBCORES = 16
NUM_WORKERS = NUM_SC_CORES * NUM_SUBCORES

BATCH = 16384
EMBED = 32
ROWS_PER_WORKER = BATCH // NUM_WORKERS
IDX_CHUNK = 128
NUM_CHUNKS = ROWS_PER_WORKER // IDX_CHUNK


def _sc_gather_one(table_pad, ids):
    mesh = plsc.VectorSubcoreMesh(core_axis_name="c", subcore_axis_name="s")

    @functools.partial(
        pl.kernel,
        mesh=mesh,
        compiler_params=pltpu.CompilerParams(use_tc_tiling_on_sc=False),
        out_type=jax.ShapeDtypeStruct((BATCH, 128), jnp.float32),
        scratch_types=[
            pltpu.VMEM((ROWS_PER_WORKER,), jnp.int32),
            pltpu.VMEM((ROWS_PER_WORKER, 128), jnp.float32),
            pltpu.SemaphoreType.DMA,
        ],
    )
    def k(t_hbm, id_hbm, out, idx_v, rows_v, sem):
        wid = lax.axis_index("s") * NUM_SC_CORES + lax.axis_index("c")
        base = wid * ROWS_PER_WORKER
        pltpu.sync_copy(id_hbm.at[pl.ds(base, ROWS_PER_WORKER)], idx_v)
        copies = []
        for j in range(NUM_CHUNKS):
            idx = pl.ds(j * IDX_CHUNK, IDX_CHUNK)
            copies.append(pltpu.async_copy(
                t_hbm.at[idx_v.at[idx]], rows_v.at[idx], sem))
        for c in copies:
            c.wait()
        pltpu.sync_copy(rows_v, out.at[pl.ds(base, ROWS_PER_WORKER)])

    return k(table_pad, ids)


def _tc_body(u_ref, i_ref, f_ref, wf_ref, bf_ref, w_ref, b_ref, out_ref):
    g = lax.dot_general(f_ref[...], wf_ref[...], (((1,), (1,)), ((), ())),
                        preferred_element_type=jnp.float32)
    u = u_ref[...][:, :EMBED]
    i = i_ref[...][:, :EMBED]
    acc = u * (i + g + bf_ref[...])
    out = jnp.sum(acc * w_ref[...], axis=1) + b_ref[0, 0]
    out_ref[...] = out[None, None, :]


def _tc_combine(u4, i4, f, Wf, bf2, W, b2):
    nblk = 8
    blk = BATCH // nblk
    return pl.pallas_call(
        _tc_body,
        grid=(nblk,),
        in_specs=[
            pl.BlockSpec((blk, 128), lambda n: (n, 0)),
            pl.BlockSpec((blk, 128), lambda n: (n, 0)),
            pl.BlockSpec((blk, 100), lambda n: (n, 0)),
            pl.BlockSpec((EMBED, 100), lambda n: (0, 0)),
            pl.BlockSpec((1, EMBED), lambda n: (0, 0)),
            pl.BlockSpec((1, EMBED), lambda n: (0, 0)),
            pl.BlockSpec((1, 1), lambda n: (0, 0)),
        ],
        out_specs=pl.BlockSpec((1, 1, blk), lambda n: (n, 0, 0)),
        out_shape=jax.ShapeDtypeStruct((nblk, 1, blk), jnp.float32),
    )(u4, i4, f, Wf, bf2, W, b2)


def kernel(user_ids, item_ids, features, user_table, item_table, Wf, bf, W, b):
    ut_pad = jnp.pad(user_table, ((0, 0), (0, 128 - EMBED)))
    it_pad = jnp.pad(item_table, ((0, 0), (0, 128 - EMBED)))
    u = _sc_gather_one(ut_pad, user_ids.astype(jnp.int32))
    i = _sc_gather_one(it_pad, item_ids.astype(jnp.int32))
    out = _tc_combine(u, i, features, Wf, bf.reshape(1, EMBED), W,
                      b.reshape(1, 1))
    return out.reshape(BATCH)
